# Initial kernel scaffold; baseline (speedup 1.0000x reference)
import jax
import jax.numpy as jnp
from jax.experimental import pallas as pl


def kernel(action_ids, table):
    raise NotImplementedError("write your pallas kernel here")


# speedup vs baseline: 1.1046x; 1.1046x over previous
import functools

import jax
import jax.numpy as jnp
from jax import lax
from jax.experimental import pallas as pl
from jax.experimental.pallas import tpu as pltpu
from jax.experimental.pallas import tpu_sc as plsc

VOCAB = 1000000
EMBED_DIM = 32
BATCH = 16384
HIST = 50

B = BATCH * HIST
NC = 2
NS = 16
NW = NC * NS
B_PER_W = B // NW
IW = 128
K = 8
CHUNK = K * IW
N_CHUNKS = B_PER_W // CHUNK
IDX_ROWS_PER_W = B_PER_W // IW


def _body(idx_hbm, table_hbm, out_hbm, idx_v, rows_v, sem0, sem1):
    wid = lax.axis_index("s") * NC + lax.axis_index("c")
    base = wid * B_PER_W
    idx_row_base = wid * IDX_ROWS_PER_W
    sems = (sem0, sem1)

    def start(g, slot):
        pltpu.sync_copy(idx_hbm.at[pl.ds(idx_row_base + g * K, K)],
                        idx_v.at[slot])
        for j in range(K):
            pltpu.async_copy(table_hbm.at[idx_v.at[slot, j]],
                             rows_v.at[slot, pl.ds(j * IW, IW)],
                             sems[slot])

    def finish(g, slot):
        pltpu.make_async_copy(table_hbm.at[pl.ds(0, CHUNK)],
                              rows_v.at[slot], sems[slot]).wait()
        pltpu.sync_copy(rows_v.at[slot],
                        out_hbm.at[pl.ds(base + g * CHUNK, CHUNK)])

    start(0, 0)

    def pair(p, _):
        g = 2 * p
        start(g + 1, 1)
        finish(g, 0)
        start(g + 2, 0)
        finish(g + 1, 1)
        return _

    lax.fori_loop(0, (N_CHUNKS - 1) // 2, pair, None)
    finish(N_CHUNKS - 1, 0)


@jax.jit
def _gather(idx_rows, table):
    mesh = plsc.VectorSubcoreMesh(core_axis_name="c", subcore_axis_name="s")
    k = functools.partial(
        pl.kernel,
        mesh=mesh,
        out_type=jax.ShapeDtypeStruct((B, EMBED_DIM), jnp.float32),
        scratch_types=[
            pltpu.VMEM((2, K, IW), jnp.int32),
            pltpu.VMEM((2, CHUNK, EMBED_DIM), jnp.float32),
            pltpu.SemaphoreType.DMA,
            pltpu.SemaphoreType.DMA,
        ],
        compiler_params=pltpu.CompilerParams(use_tc_tiling_on_sc=False),
    )(_body)
    return k(idx_rows, table)


def kernel(action_ids, table):
    out = _gather(action_ids.reshape(B // IW, IW), table)
    return out.reshape(BATCH, HIST, EMBED_DIM)


# speedup vs baseline: 1.8018x; 1.6312x over previous
import functools

import jax
import jax.numpy as jnp
from jax import lax
from jax.experimental import pallas as pl
from jax.experimental.pallas import tpu as pltpu
from jax.experimental.pallas import tpu_sc as plsc

VOCAB = 1000000
EMBED_DIM = 32
BATCH = 16384
HIST = 50

NC = 2
NS = 16
NW = NC * NS
ROWS_PER_W = BATCH // NW
CHUNK_R = 32
N_CHUNKS = ROWS_PER_W // CHUNK_R


def _body(idx_hbm, table_hbm, out_hbm, idx_v, rows_v, sem0, sem1):
    wid = lax.axis_index("s") * NC + lax.axis_index("c")
    base = wid * ROWS_PER_W
    sems = (sem0, sem1)

    def start(g, slot):
        r0 = base + g * CHUNK_R
        pltpu.sync_copy(idx_hbm.at[pl.ds(r0, CHUNK_R)], idx_v.at[slot])

        def gather_row(j, _):
            pltpu.async_copy(table_hbm.at[idx_v.at[slot, j]],
                             rows_v.at[slot, j], sems[slot])
            return _

        lax.fori_loop(0, CHUNK_R, gather_row, None)

    def finish(g, slot):
        r0 = base + g * CHUNK_R
        pltpu.make_async_copy(out_hbm.at[pl.ds(0, CHUNK_R)],
                              rows_v.at[slot], sems[slot]).wait()
        pltpu.sync_copy(rows_v.at[slot], out_hbm.at[pl.ds(r0, CHUNK_R)])

    start(0, 0)

    def pair(p, _):
        g = 2 * p
        start(g + 1, 1)
        finish(g, 0)
        start(g + 2, 0)
        finish(g + 1, 1)
        return _

    lax.fori_loop(0, N_CHUNKS // 2 - 1, pair, None)
    g = N_CHUNKS - 2
    start(g + 1, 1)
    finish(g, 0)
    finish(g + 1, 1)


@jax.jit
def _gather(action_ids, table):
    mesh = plsc.VectorSubcoreMesh(core_axis_name="c", subcore_axis_name="s")
    k = functools.partial(
        pl.kernel,
        mesh=mesh,
        out_type=jax.ShapeDtypeStruct((BATCH, HIST, EMBED_DIM), jnp.float32),
        scratch_types=[
            pltpu.VMEM((2, CHUNK_R, HIST), jnp.int32),
            pltpu.VMEM((2, CHUNK_R, HIST, EMBED_DIM), jnp.float32),
            pltpu.SemaphoreType.DMA,
            pltpu.SemaphoreType.DMA,
        ],
        compiler_params=pltpu.CompilerParams(use_tc_tiling_on_sc=False),
    )(_body)
    return k(action_ids, table)


def kernel(action_ids, table):
    return _gather(action_ids, table)
